# baseline (speedup 1.0000x reference)
import functools

import jax
import jax.numpy as jnp
from jax import lax
from jax.experimental import pallas as pl
from jax.experimental.pallas import tpu as pltpu
from jax.experimental.pallas import tpu_sc as plsc

_NC = 2
_NS = 16
_NW = _NC * _NS
_L = 16

_BOUNDS = (0.1, 0.2, 0.3, 0.4, 0.5, 0.6, 0.7, 0.8, 0.9)

_CHUNK = 25600
_NBUF = 4


@functools.cache
def _make_sc_call(n):
    assert n % (_NW * _L) == 0, n
    per_w = n // _NW
    chunk = min(_CHUNK, per_w)
    assert per_w % chunk == 0, (per_w, chunk)
    n_vec = chunk // _L
    mesh = plsc.VectorSubcoreMesh(
        core_axis_name="c", subcore_axis_name="s",
        num_cores=_NC, num_subcores=_NS,
    )

    @functools.partial(
        pl.kernel,
        out_type=jax.ShapeDtypeStruct((2 * n,), jnp.float32),
        mesh=mesh,
        compiler_params=pltpu.CompilerParams(needs_layout_passes=False),
        scratch_types=[
            [pltpu.VMEM((chunk,), jnp.float32)] * _NBUF,
            [pltpu.VMEM((_L,), jnp.float32)] * 3,
            [pltpu.SemaphoreType.DMA] * _NBUF,
            [pltpu.SemaphoreType.DMA] * _NBUF,
            pltpu.SemaphoreType.DMA,
        ],
    )
    def sc_fn(w1_hbm, w2_hbm, bw1_hbm, bw2_hbm, extb_hbm, out_hbm,
              bufs, tables, sins, souts, sbw):
        wid = lax.axis_index("c") * _NS + lax.axis_index("s")
        base = wid * per_w
        bwv1, bwv2, extb = tables

        bw_copy1 = pltpu.async_copy(bw1_hbm, bwv1, sbw)
        bw_copy2 = pltpu.async_copy(bw2_hbm, bwv2, sbw)
        bw_copy3 = pltpu.async_copy(extb_hbm, extb, sbw)

        tasks = []
        for src, out_off, bw in ((w1_hbm, 0, bwv1), (w2_hbm, n, bwv2)):
            for c in range(per_w // chunk):
                off = base + c * chunk
                tasks.append((src, off, out_off + off, bw))
        T = len(tasks)

        def start_load(t):
            src, off, _, _ = tasks[t]
            return pltpu.async_copy(
                src.at[pl.ds(off, chunk)], bufs[t % _NBUF], sins[t % _NBUF])

        def start_store(t):
            _, _, dst_off, _ = tasks[t]
            return pltpu.async_copy(
                bufs[t % _NBUF], out_hbm.at[pl.ds(dst_off, chunk)],
                souts[t % _NBUF])

        pending = {}
        for t in range(min(_NBUF, T)):
            pending[t] = start_load(t)
        bw_copy1.wait()
        bw_copy2.wait()
        bw_copy3.wait()
        store_pending = {}
        lead = max(_NBUF - 2, 1)
        for t in range(T):
            b = t % _NBUF
            pending.pop(t).wait()
            buf = bufs[b]
            bw = tasks[t][3]

            @plsc.parallel_loop(0, n_vec, 1, unroll=8)
            def _(i):
                v = buf[pl.ds(i * _L, _L)]
                j = jnp.clip((v * 10.0).astype(jnp.int32), 0, 9)
                g = plsc.load_gather(extb, [j])
                cnt = j - (v <= g).astype(jnp.int32)
                buf[pl.ds(i * _L, _L)] = plsc.load_gather(bw, [cnt])

            store_pending[t] = start_store(t)
            u = t + lead
            if _NBUF <= u < T:
                store_pending.pop(u - _NBUF).wait()
                pending[u] = start_load(u)
        for t in sorted(store_pending):
            store_pending.pop(t).wait()

    return sc_fn


def kernel(weights_f1, weights_f2, bucket_w_f1, bucket_w_f2):
    n = weights_f1.shape[0]
    pad = jnp.zeros((_L - bucket_w_f1.shape[0],), jnp.float32)
    bw1 = jnp.concatenate([bucket_w_f1, pad])
    bw2 = jnp.concatenate([bucket_w_f2, pad])
    extb = jnp.concatenate([
        jnp.array([-1e30], jnp.float32),
        jnp.array(_BOUNDS, jnp.float32),
        jnp.zeros((_L - 10,), jnp.float32),
    ])
    return _make_sc_call(n)(weights_f1, weights_f2, bw1, bw2, extb)


# speedup vs baseline: 1.0366x; 1.0366x over previous
import functools

import jax
import jax.numpy as jnp
from jax import lax
from jax.experimental import pallas as pl
from jax.experimental.pallas import tpu as pltpu
from jax.experimental.pallas import tpu_sc as plsc

_NC = 2
_NS = 16
_NW = _NC * _NS
_L = 16

_BOUNDS = (0.1, 0.2, 0.3, 0.4, 0.5, 0.6, 0.7, 0.8, 0.9)

_CHUNK = 25600
_NBUF = 2


@functools.cache
def _make_sc_call(n):
    assert n % (_NW * _L) == 0, n
    per_w = n // _NW
    chunk = min(_CHUNK, per_w)
    assert per_w % chunk == 0, (per_w, chunk)
    n_vec = chunk // _L
    mesh = plsc.VectorSubcoreMesh(
        core_axis_name="c", subcore_axis_name="s",
        num_cores=_NC, num_subcores=_NS,
    )

    @functools.partial(
        pl.kernel,
        out_type=jax.ShapeDtypeStruct((2 * n,), jnp.float32),
        mesh=mesh,
        compiler_params=pltpu.CompilerParams(needs_layout_passes=False),
        scratch_types=[
            [pltpu.VMEM((chunk,), jnp.float32)] * _NBUF,
            [pltpu.VMEM((chunk,), jnp.float32)] * _NBUF,
            [pltpu.VMEM((_L,), jnp.float32)] * 3,
            [pltpu.SemaphoreType.DMA] * _NBUF,
            [pltpu.SemaphoreType.DMA] * _NBUF,
            pltpu.SemaphoreType.DMA,
        ],
    )
    def sc_fn(w1_hbm, w2_hbm, bw1_hbm, bw2_hbm, extb_hbm, out_hbm,
              ins, outs, tables, sins, souts, sbw):
        wid = lax.axis_index("c") * _NS + lax.axis_index("s")
        base = wid * per_w
        bwv1, bwv2, extb = tables

        bw_copy1 = pltpu.async_copy(bw1_hbm, bwv1, sbw)
        bw_copy2 = pltpu.async_copy(bw2_hbm, bwv2, sbw)
        bw_copy3 = pltpu.async_copy(extb_hbm, extb, sbw)

        tasks = []
        for src, out_off, bw in ((w1_hbm, 0, bwv1), (w2_hbm, n, bwv2)):
            for c in range(per_w // chunk):
                off = base + c * chunk
                tasks.append((src, off, out_off + off, bw))
        T = len(tasks)

        def start_load(t):
            src, off, _, _ = tasks[t]
            return pltpu.async_copy(
                src.at[pl.ds(off, chunk)], ins[t % _NBUF], sins[t % _NBUF])

        def start_store(t):
            _, _, dst_off, _ = tasks[t]
            return pltpu.async_copy(
                outs[t % _NBUF], out_hbm.at[pl.ds(dst_off, chunk)],
                souts[t % _NBUF])

        pending = {}
        for t in range(min(_NBUF, T)):
            pending[t] = start_load(t)
        bw_copy1.wait()
        bw_copy2.wait()
        bw_copy3.wait()
        store_pending = {}
        for t in range(T):
            b = t % _NBUF
            pending.pop(t).wait()
            if t - _NBUF >= 0:
                store_pending.pop(t - _NBUF).wait()
            inb, outb = ins[b], outs[b]
            bw = tasks[t][3]

            @plsc.parallel_loop(0, n_vec, 1, unroll=8)
            def _(i):
                v = inb[pl.ds(i * _L, _L)]
                j = jnp.clip((v * 10.0).astype(jnp.int32), 0, 9)
                g = plsc.load_gather(extb, [j])
                cnt = j - (v <= g).astype(jnp.int32)
                outb[pl.ds(i * _L, _L)] = plsc.load_gather(bw, [cnt])

            store_pending[t] = start_store(t)
            if t + _NBUF < T:
                pending[t + _NBUF] = start_load(t + _NBUF)
        for t in sorted(store_pending):
            store_pending.pop(t).wait()

    return sc_fn


def kernel(weights_f1, weights_f2, bucket_w_f1, bucket_w_f2):
    n = weights_f1.shape[0]
    pad = jnp.zeros((_L - bucket_w_f1.shape[0],), jnp.float32)
    bw1 = jnp.concatenate([bucket_w_f1, pad])
    bw2 = jnp.concatenate([bucket_w_f2, pad])
    extb = jnp.concatenate([
        jnp.array([-1e30], jnp.float32),
        jnp.array(_BOUNDS, jnp.float32),
        jnp.zeros((_L - 10,), jnp.float32),
    ])
    return _make_sc_call(n)(weights_f1, weights_f2, bw1, bw2, extb)
